# baseline (speedup 1.0000x reference)
import functools

import jax
import jax.numpy as jnp
from jax import lax
from jax.experimental import pallas as pl
from jax.experimental.pallas import tpu as pltpu
from jax.experimental.pallas import tpu_sc as plsc

D = 64
SCALE = 8.0
NBUF = 4
BT = 4

_info = plsc.get_sparse_core_info()
NC, NS, L = _info.num_cores, _info.num_subcores, _info.num_lanes
NW = NC * NS


def _sc_gather(emb, tok4):
    sb, tbn = tok4.shape[0], tok4.shape[1]
    seq = sb * 8
    nt = seq // NBUF
    unit_rows = 128

    mesh = plsc.VectorSubcoreMesh(core_axis_name="c", subcore_axis_name="s")

    @functools.partial(
        pl.kernel,
        mesh=mesh,
        compiler_params=pltpu.CompilerParams(
            use_tc_tiling_on_sc=False, needs_layout_passes=False),
        out_type=jax.ShapeDtypeStruct((seq * tbn * unit_rows, D),
                                      jnp.float32),
        scratch_types=(
            [pltpu.VMEM((sb, 8, 128), jnp.int32)]
            + [pltpu.VMEM((128, D), jnp.float32) for _ in range(NBUF)]
            + [pltpu.VMEM((128, D), jnp.float32) for _ in range(NBUF)]
            + [pltpu.SemaphoreType.DMA for _ in range(2 * NBUF)]
        ),
    )
    def k(emb_hbm, tok_hbm, out_hbm, idx_v, *bufs_and_sems):
        a_bufs = bufs_and_sems[:NBUF]
        b_bufs = bufs_and_sems[NBUF:2 * NBUF]
        gsems = bufs_and_sems[2 * NBUF:3 * NBUF]
        ssems = bufs_and_sems[3 * NBUF:]

        wid = lax.axis_index("s") * NC + lax.axis_index("c")
        for a in range(sb):
            pltpu.sync_copy(tok_hbm.at[a, wid], idx_v.at[a])

        def row0(s):
            return (s * tbn + wid) * unit_rows

        def fire_gather(b, s):
            pltpu.async_copy(
                emb_hbm.at[idx_v.at[s // 8, s % 8]], a_bufs[b], gsems[b])

        def wait_gather(b, s):
            pltpu.make_async_copy(
                emb_hbm.at[idx_v.at[s // 8, s % 8]], a_bufs[b],
                gsems[b]).wait()

        def fire_store(b, s):
            pltpu.async_copy(
                b_bufs[b], out_hbm.at[pl.ds(row0(s), unit_rows)], ssems[b])

        def wait_store(b, s):
            pltpu.make_async_copy(
                b_bufs[b], out_hbm.at[pl.ds(row0(s), unit_rows)],
                ssems[b]).wait()

        def scale(b):
            src, dst = a_bufs[b], b_bufs[b]

            def rows(i, _):
                base = i * 4
                for rr in range(4):
                    for j in range(D // L):
                        dst[base + rr, pl.ds(j * L, L)] = (
                            src[base + rr, pl.ds(j * L, L)] * SCALE)
                return 0

            lax.fori_loop(0, 32, rows, 0)

        for b in range(NBUF):
            fire_gather(b, b)

        for b in range(NBUF):
            wait_gather(b, b)
            scale(b)
            fire_gather(b, NBUF + b)
            fire_store(b, b)

        def step(t, _):
            for b in range(NBUF):
                s = t * NBUF + b
                wait_gather(b, s)
                wait_store(b, s - NBUF)
                scale(b)
                fire_gather(b, s + NBUF)
                fire_store(b, s)
            return 0

        lax.fori_loop(1, nt - 1, step, 0)

        for b in range(NBUF):
            s = (nt - 1) * NBUF + b
            wait_gather(b, s)
            wait_store(b, s - NBUF)
            scale(b)
            fire_store(b, s)

        for b in range(NBUF):
            wait_store(b, (nt - 1) * NBUF + b)

    return k(emb, tok4)


def _tc_transpose(x, seq, tbn):
    def body(x_ref, y_ref):
        for u in range(BT):
            xu = x_ref[pl.ds(u * 128, 128), :]
            y_ref[0, :, u] = jnp.swapaxes(xu, 0, 1).reshape(8, 8, 128)

    return pl.pallas_call(
        body,
        grid=(seq, tbn // BT),
        in_specs=[pl.BlockSpec((128 * BT, D),
                               lambda s, j: (s * (tbn // BT) + j, 0))],
        out_specs=pl.BlockSpec((1, 8, BT, 8, 128),
                               lambda s, j: (s, 0, j, 0, 0)),
        out_shape=jax.ShapeDtypeStruct((seq, 8, tbn, 8, 128), jnp.float32),
    )(x)


def kernel(tokens, embedding):
    bsz, seq = tokens.shape
    sb, tbn = seq // 8, bsz // 128
    tok4 = (tokens.astype(jnp.int32).T
            .reshape(sb, 8, tbn, 128).transpose(0, 2, 1, 3))
    x = _sc_gather(embedding, tok4)
    y5 = _tc_transpose(x, seq, tbn)
    return jnp.transpose(y5, (2, 4, 0, 1, 3)).reshape(bsz, seq, D)


# speedup vs baseline: 1.0962x; 1.0962x over previous
import functools

import jax
import jax.numpy as jnp
from jax import lax
from jax.experimental import pallas as pl
from jax.experimental.pallas import tpu as pltpu
from jax.experimental.pallas import tpu_sc as plsc

D = 64
SCALE = 8.0
NBUF = 4
BT = 4
UR = 64

_info = plsc.get_sparse_core_info()
NC, NS, L = _info.num_cores, _info.num_subcores, _info.num_lanes
NW = NC * NS


def _sc_gather(emb, tok4):
    sb, tbn = tok4.shape[0], tok4.shape[1]
    seq = sb * 8
    nt = seq // NBUF

    mesh = plsc.VectorSubcoreMesh(core_axis_name="c", subcore_axis_name="s")

    @functools.partial(
        pl.kernel,
        mesh=mesh,
        compiler_params=pltpu.CompilerParams(
            use_tc_tiling_on_sc=False, needs_layout_passes=False),
        out_type=jax.ShapeDtypeStruct((seq * tbn * UR, 128), jnp.float32),
        scratch_types=(
            [pltpu.VMEM((sb, 8, 128), jnp.int32)]
            + [pltpu.VMEM((128,), jnp.int32) for _ in range(NBUF)]
            + [pltpu.VMEM((128, D), jnp.float32) for _ in range(NBUF)]
            + [pltpu.VMEM((UR, 128), jnp.float32) for _ in range(NBUF)]
            + [pltpu.SemaphoreType.DMA for _ in range(2 * NBUF)]
        ),
    )
    def k(emb_hbm, tok_hbm, out_hbm, idx_v, *bufs_and_sems):
        p_bufs = bufs_and_sems[:NBUF]
        a_bufs = bufs_and_sems[NBUF:2 * NBUF]
        b_bufs = bufs_and_sems[2 * NBUF:3 * NBUF]
        gsems = bufs_and_sems[3 * NBUF:4 * NBUF]
        ssems = bufs_and_sems[4 * NBUF:]

        wid = lax.axis_index("s") * NC + lax.axis_index("c")
        for a in range(sb):
            pltpu.sync_copy(tok_hbm.at[a, wid], idx_v.at[a])

        iota2 = lax.iota(jnp.int32, 16) * 2

        def row0(s):
            return (s * tbn + wid) * UR

        def fire_gather(b, s):
            for c in range(4):
                v = idx_v[s // 8, s % 8, pl.ds(c * 16, 16)]
                plsc.store_scatter(p_bufs[b], [iota2 + (c * 32)], v)
                v = idx_v[s // 8, s % 8, pl.ds(64 + c * 16, 16)]
                plsc.store_scatter(p_bufs[b], [iota2 + (c * 32 + 1)], v)
            pltpu.async_copy(emb_hbm.at[p_bufs[b]], a_bufs[b], gsems[b])

        def wait_gather(b):
            pltpu.make_async_copy(
                emb_hbm.at[p_bufs[b]], a_bufs[b], gsems[b]).wait()

        def fire_store(b, s):
            pltpu.async_copy(
                b_bufs[b], out_hbm.at[pl.ds(row0(s), UR)], ssems[b])

        def wait_store(b, s):
            pltpu.make_async_copy(
                b_bufs[b], out_hbm.at[pl.ds(row0(s), UR)], ssems[b]).wait()

        def scale(b):
            src, dst = a_bufs[b], b_bufs[b]

            def rows(i, _):
                for cc in range(8):
                    dst[i, pl.ds(cc * L, L)] = (
                        src[2 * i + cc // 4, pl.ds((cc % 4) * L, L)] * SCALE)
                return 0

            lax.fori_loop(0, UR, rows, 0)

        for b in range(NBUF):
            fire_gather(b, b)

        for b in range(NBUF):
            wait_gather(b)
            scale(b)
            fire_gather(b, NBUF + b)
            fire_store(b, b)

        def step(t, _):
            for b in range(NBUF):
                s = t * NBUF + b
                wait_gather(b)
                wait_store(b, s - NBUF)
                scale(b)
                fire_gather(b, s + NBUF)
                fire_store(b, s)
            return 0

        lax.fori_loop(1, nt - 1, step, 0)

        for b in range(NBUF):
            s = (nt - 1) * NBUF + b
            wait_gather(b)
            wait_store(b, s - NBUF)
            scale(b)
            fire_store(b, s)

        for b in range(NBUF):
            wait_store(b, (nt - 1) * NBUF + b)

    return k(emb, tok4)


def _tc_transpose(x, seq, tbn):
    def body(x_ref, y_ref):
        for u in range(BT):
            xu = x_ref[pl.ds(u * UR, UR), :]
            lo = jnp.swapaxes(xu[:, :64], 0, 1)
            hi = jnp.swapaxes(xu[:, 64:], 0, 1)
            y = jnp.concatenate([lo, hi], axis=1)
            y_ref[0, :, u] = y.reshape(8, 8, 128)

    return pl.pallas_call(
        body,
        grid=(seq, tbn // BT),
        in_specs=[pl.BlockSpec((UR * BT, 128),
                               lambda s, j: (s * (tbn // BT) + j, 0))],
        out_specs=pl.BlockSpec((1, 8, BT, 8, 128),
                               lambda s, j: (s, 0, j, 0, 0)),
        out_shape=jax.ShapeDtypeStruct((seq, 8, tbn, 8, 128), jnp.float32),
    )(x)


def kernel(tokens, embedding):
    bsz, seq = tokens.shape
    sb, tbn = seq // 8, bsz // 128
    tok4 = (tokens.astype(jnp.int32).T
            .reshape(sb, 8, tbn, 128).transpose(0, 2, 1, 3))
    x = _sc_gather(embedding, tok4)
    y5 = _tc_transpose(x, seq, tbn)
    return jnp.transpose(y5, (2, 4, 0, 1, 3)).reshape(bsz, seq, D)


# speedup vs baseline: 1.2338x; 1.1254x over previous
import functools

import jax
import jax.numpy as jnp
from jax import lax
from jax.experimental import pallas as pl
from jax.experimental.pallas import tpu as pltpu
from jax.experimental.pallas import tpu_sc as plsc

D = 64
SCALE = 8.0
NBUF = 4
BT = 4
UR = 64

_info = plsc.get_sparse_core_info()
NC, NS, L = _info.num_cores, _info.num_subcores, _info.num_lanes
NW = NC * NS


def _sc_gather(emb, tok4):
    sb, tbn = tok4.shape[0], tok4.shape[1]
    seq = sb * 8
    nt = seq // NBUF

    mesh = plsc.VectorSubcoreMesh(core_axis_name="c", subcore_axis_name="s")

    @functools.partial(
        pl.kernel,
        mesh=mesh,
        compiler_params=pltpu.CompilerParams(
            use_tc_tiling_on_sc=False, needs_layout_passes=False),
        out_type=jax.ShapeDtypeStruct((seq * tbn * UR, 128), jnp.float32),
        scratch_types=(
            [pltpu.VMEM((sb, 8, 128), jnp.int32)]
            + [pltpu.VMEM((128,), jnp.int32) for _ in range(NBUF)]
            + [pltpu.VMEM((128, D), jnp.float32) for _ in range(NBUF)]
            + [pltpu.VMEM((UR, 128), jnp.float32) for _ in range(NBUF)]
            + [pltpu.SemaphoreType.DMA for _ in range(2 * NBUF)]
        ),
    )
    def k(emb_hbm, tok_hbm, out_hbm, idx_v, *bufs_and_sems):
        p_bufs = bufs_and_sems[:NBUF]
        a_bufs = bufs_and_sems[NBUF:2 * NBUF]
        b_bufs = bufs_and_sems[2 * NBUF:3 * NBUF]
        gsems = bufs_and_sems[3 * NBUF:4 * NBUF]
        ssems = bufs_and_sems[4 * NBUF:]

        wid = lax.axis_index("s") * NC + lax.axis_index("c")
        for a in range(sb):
            pltpu.sync_copy(tok_hbm.at[a, wid], idx_v.at[a])

        iota2 = lax.iota(jnp.int32, 16) * 2

        def row0(s):
            return (s * tbn + wid) * UR

        def fire_gather(b, s):
            for c in range(4):
                v = idx_v[s // 8, s % 8, pl.ds(c * 16, 16)]
                plsc.store_scatter(p_bufs[b], [iota2 + (c * 32)], v)
                v = idx_v[s // 8, s % 8, pl.ds(64 + c * 16, 16)]
                plsc.store_scatter(p_bufs[b], [iota2 + (c * 32 + 1)], v)
            pltpu.async_copy(emb_hbm.at[p_bufs[b]], a_bufs[b], gsems[b])

        def wait_gather(b):
            pltpu.make_async_copy(
                emb_hbm.at[p_bufs[b]], a_bufs[b], gsems[b]).wait()

        def fire_store(b, s):
            pltpu.async_copy(
                b_bufs[b], out_hbm.at[pl.ds(row0(s), UR)], ssems[b])

        def wait_store(b, s):
            pltpu.make_async_copy(
                b_bufs[b], out_hbm.at[pl.ds(row0(s), UR)], ssems[b]).wait()

        def scale(b):
            src, dst = a_bufs[b], b_bufs[b]

            def rows(i, _):
                for cc in range(8):
                    dst[i, pl.ds(cc * L, L)] = (
                        src[2 * i + cc // 4, pl.ds((cc % 4) * L, L)] * SCALE)
                return 0

            lax.fori_loop(0, UR, rows, 0)

        for b in range(NBUF):
            fire_gather(b, b)

        for b in range(NBUF):
            wait_gather(b)
            scale(b)
            fire_gather(b, NBUF + b)
            fire_store(b, b)

        def step(t, _):
            for b in range(NBUF):
                s = t * NBUF + b
                wait_gather(b)
                wait_store(b, s - NBUF)
                scale(b)
                fire_gather(b, s + NBUF)
                fire_store(b, s)
            return 0

        lax.fori_loop(1, nt - 1, step, 0)

        for b in range(NBUF):
            s = (nt - 1) * NBUF + b
            wait_gather(b)
            wait_store(b, s - NBUF)
            scale(b)
            fire_store(b, s)

        for b in range(NBUF):
            wait_store(b, (nt - 1) * NBUF + b)

    return k(emb, tok4)


def _tc_transpose(x, seq, tbn):
    def body(x_ref, y_ref):
        for u in range(tbn):
            xu = x_ref[pl.ds(u * UR, UR), :]
            lo = jnp.swapaxes(xu[:, :64], 0, 1)
            hi = jnp.swapaxes(xu[:, 64:], 0, 1)
            y = jnp.concatenate([lo, hi], axis=1)
            y_ref[0, :, u] = y.reshape(8, 8, 128)

    return pl.pallas_call(
        body,
        grid=(seq,),
        in_specs=[pl.BlockSpec((UR * tbn, 128), lambda s: (s, 0))],
        out_specs=pl.BlockSpec((1, 8, tbn, 8, 128),
                               lambda s: (s, 0, 0, 0, 0)),
        out_shape=jax.ShapeDtypeStruct((seq, 8, tbn, 8, 128), jnp.float32),
    )(x)


def kernel(tokens, embedding):
    bsz, seq = tokens.shape
    sb, tbn = seq // 8, bsz // 128
    tok4 = (tokens.astype(jnp.int32).T
            .reshape(sb, 8, tbn, 128).transpose(0, 2, 1, 3))
    x = _sc_gather(embedding, tok4)
    y5 = _tc_transpose(x, seq, tbn)
    return jnp.transpose(y5, (2, 4, 0, 1, 3)).reshape(bsz, seq, D)


# speedup vs baseline: 1.6516x; 1.3387x over previous
import functools

import jax
import jax.numpy as jnp
from jax import lax
from jax.experimental import pallas as pl
from jax.experimental.pallas import tpu as pltpu
from jax.experimental.pallas import tpu_sc as plsc

D = 64
G = 128
SCALE = 8.0
NBUF = 4
RU = 4

_info = plsc.get_sparse_core_info()
NC, NS, L = _info.num_cores, _info.num_subcores, _info.num_lanes
NW = NC * NS


def _gather_scaled(emb, idx2d):
    ng_total = idx2d.shape[0]
    ng_per_w = ng_total // NW
    b_total = ng_total * G
    nt = ng_per_w // NBUF

    mesh = plsc.VectorSubcoreMesh(core_axis_name="c", subcore_axis_name="s")

    @functools.partial(
        pl.kernel,
        mesh=mesh,
        compiler_params=pltpu.CompilerParams(use_tc_tiling_on_sc=False),
        out_type=jax.ShapeDtypeStruct((b_total, D), jnp.float32),
        scratch_types=(
            [pltpu.VMEM((ng_per_w, G), jnp.int32)]
            + [pltpu.VMEM((G, D), jnp.float32) for _ in range(2 * NBUF)]
            + [pltpu.SemaphoreType.DMA for _ in range(2 * NBUF)]
        ),
    )
    def k(emb_hbm, idx_hbm, out_hbm, idx_v, *bufs_and_sems):
        a_bufs = bufs_and_sems[:NBUF]
        b_bufs = bufs_and_sems[NBUF:2 * NBUF]
        gsems = bufs_and_sems[2 * NBUF:3 * NBUF]
        ssems = bufs_and_sems[3 * NBUF:]

        wid = lax.axis_index("s") * NC + lax.axis_index("c")
        g0 = wid * ng_per_w
        pltpu.sync_copy(idx_hbm.at[pl.ds(g0, ng_per_w)], idx_v)

        def fire_gather(b, g):
            pltpu.async_copy(emb_hbm.at[idx_v.at[g]], a_bufs[b], gsems[b])

        def wait_gather(b, g):
            pltpu.make_async_copy(
                emb_hbm.at[idx_v.at[g]], a_bufs[b], gsems[b]).wait()

        def fire_store(b, g):
            pltpu.async_copy(
                b_bufs[b], out_hbm.at[pl.ds((g0 + g) * G, G)], ssems[b])

        def wait_store(b, g):
            pltpu.make_async_copy(
                b_bufs[b], out_hbm.at[pl.ds((g0 + g) * G, G)], ssems[b]).wait()

        def scale(b):
            src, dst = a_bufs[b], b_bufs[b]

            def rows(i, _):
                base = i * RU
                for rr in range(RU):
                    for j in range(D // L):
                        dst[base + rr, pl.ds(j * L, L)] = (
                            src[base + rr, pl.ds(j * L, L)] * SCALE)
                return 0

            lax.fori_loop(0, G // RU, rows, 0)

        for b in range(NBUF):
            fire_gather(b, b)

        for b in range(NBUF):
            wait_gather(b, b)
            scale(b)
            fire_gather(b, NBUF + b)
            fire_store(b, b)

        def step(t, _):
            for b in range(NBUF):
                g = t * NBUF + b
                wait_gather(b, g)
                wait_store(b, g - NBUF)
                scale(b)
                fire_gather(b, g + NBUF)
                fire_store(b, g)
            return 0

        lax.fori_loop(1, nt - 1, step, 0)

        for b in range(NBUF):
            g = (nt - 1) * NBUF + b
            wait_gather(b, g)
            wait_store(b, g - NBUF)
            scale(b)
            fire_store(b, g)

        for b in range(NBUF):
            wait_store(b, (nt - 1) * NBUF + b)

    return k(emb, idx2d)


def kernel(tokens, embedding):
    b, s = tokens.shape
    idx = tokens.astype(jnp.int32).reshape(b * s // G, G)
    out = _gather_scaled(embedding, idx)
    return out.reshape(b, s, D)


# speedup vs baseline: 1.7834x; 1.0798x over previous
import functools

import jax
import jax.numpy as jnp
from jax import lax
from jax.experimental import pallas as pl
from jax.experimental.pallas import tpu as pltpu
from jax.experimental.pallas import tpu_sc as plsc

D = 64
SCALE = 8.0
NBUF = 4
PAD = 130

_info = plsc.get_sparse_core_info()
NC, NS, L = _info.num_cores, _info.num_subcores, _info.num_lanes
NW = NC * NS


def _lookup(emb, tok4):
    sb, tbn = tok4.shape[0], tok4.shape[1]
    seq = sb * 8
    nt = seq // NBUF

    mesh = plsc.VectorSubcoreMesh(core_axis_name="c", subcore_axis_name="s")

    @functools.partial(
        pl.kernel,
        mesh=mesh,
        compiler_params=pltpu.CompilerParams(
            use_tc_tiling_on_sc=False, needs_layout_passes=False),
        out_type=jax.ShapeDtypeStruct((seq, D // 8, tbn, 8, 128), jnp.float32),
        scratch_types=(
            [pltpu.VMEM((sb, 8, 128), jnp.int32)]
            + [pltpu.VMEM((128, D), jnp.float32) for _ in range(NBUF)]
            + [pltpu.VMEM((D // 8, 8, PAD), jnp.float32) for _ in range(NBUF)]
            + [pltpu.SemaphoreType.DMA for _ in range(2 * NBUF)]
        ),
    )
    def k(emb_hbm, tok_hbm, out_hbm, idx_v, *bufs_and_sems):
        a_bufs = bufs_and_sems[:NBUF]
        b_bufs = bufs_and_sems[NBUF:2 * NBUF]
        gsems = bufs_and_sems[2 * NBUF:3 * NBUF]
        ssems = bufs_and_sems[3 * NBUF:]

        wid = lax.axis_index("s") * NC + lax.axis_index("c")
        for a in range(sb):
            pltpu.sync_copy(tok_hbm.at[a, wid], idx_v.at[a])

        iota = lax.iota(jnp.int32, 16)
        tds = [(iota + c * 16) // 8 for c in range(D // L)]
        rds = [(iota + c * 16) % 8 for c in range(D // L)]

        def fire_gather(b, s):
            pltpu.async_copy(
                emb_hbm.at[idx_v.at[s // 8, s % 8]], a_bufs[b], gsems[b])

        def wait_gather(b, s):
            pltpu.make_async_copy(
                emb_hbm.at[idx_v.at[s // 8, s % 8]], a_bufs[b],
                gsems[b]).wait()

        def fire_store(b, s):
            pltpu.async_copy(
                b_bufs[b].at[:, :, pl.ds(0, 128)],
                out_hbm.at[s, :, wid], ssems[b])

        def wait_store(b, s):
            pltpu.make_async_copy(
                b_bufs[b].at[:, :, pl.ds(0, 128)],
                out_hbm.at[s, :, wid], ssems[b]).wait()

        def transpose_scale(b):
            src, dst = a_bufs[b], b_bufs[b]

            def rows(i, _):
                rb = lax.broadcast(i, (16,))
                for c in range(D // L):
                    v = src[i, pl.ds(c * L, L)] * SCALE
                    plsc.store_scatter(dst, [tds[c], rds[c], rb], v)
                return 0

            lax.fori_loop(0, 128, rows, 0)

        for b in range(NBUF):
            fire_gather(b, b)

        for b in range(NBUF):
            wait_gather(b, b)
            transpose_scale(b)
            fire_gather(b, NBUF + b)
            fire_store(b, b)

        def step(t, _):
            for b in range(NBUF):
                s = t * NBUF + b
                wait_gather(b, s)
                wait_store(b, s - NBUF)
                transpose_scale(b)
                fire_gather(b, s + NBUF)
                fire_store(b, s)
            return 0

        lax.fori_loop(1, nt - 1, step, 0)

        for b in range(NBUF):
            s = (nt - 1) * NBUF + b
            wait_gather(b, s)
            wait_store(b, s - NBUF)
            transpose_scale(b)
            fire_store(b, s)

        for b in range(NBUF):
            wait_store(b, (nt - 1) * NBUF + b)

    return k(emb, tok4)


def kernel(tokens, embedding):
    bsz, seq = tokens.shape
    sb, tbn = seq // 8, bsz // 128
    tok4 = (tokens.astype(jnp.int32).T
            .reshape(sb, 8, tbn, 128).transpose(0, 2, 1, 3))
    y5 = _lookup(embedding, tok4)
    return jnp.transpose(y5, (2, 4, 0, 1, 3)).reshape(bsz, seq, D)


# speedup vs baseline: 1.8066x; 1.0130x over previous
import functools

import jax
import jax.numpy as jnp
from jax import lax
from jax.experimental import pallas as pl
from jax.experimental.pallas import tpu as pltpu
from jax.experimental.pallas import tpu_sc as plsc

D = 64
SCALE = 8.0
NBUF = 4
PAD = 129

_info = plsc.get_sparse_core_info()
NC, NS, L = _info.num_cores, _info.num_subcores, _info.num_lanes
NW = NC * NS


def _lookup(emb, tok4):
    sb, tbn = tok4.shape[0], tok4.shape[1]
    seq = sb * 8
    nt = seq // NBUF

    mesh = plsc.VectorSubcoreMesh(core_axis_name="c", subcore_axis_name="s")

    @functools.partial(
        pl.kernel,
        mesh=mesh,
        compiler_params=pltpu.CompilerParams(
            use_tc_tiling_on_sc=False, needs_layout_passes=False),
        out_type=jax.ShapeDtypeStruct((seq, D // 8, tbn, 8, 128), jnp.float32),
        scratch_types=(
            [pltpu.VMEM((sb, 8, 128), jnp.int32)]
            + [pltpu.VMEM((128, D), jnp.float32) for _ in range(NBUF)]
            + [pltpu.VMEM((D // 8, 8, PAD), jnp.float32) for _ in range(NBUF)]
            + [pltpu.SemaphoreType.DMA for _ in range(2 * NBUF)]
        ),
    )
    def k(emb_hbm, tok_hbm, out_hbm, idx_v, *bufs_and_sems):
        a_bufs = bufs_and_sems[:NBUF]
        b_bufs = bufs_and_sems[NBUF:2 * NBUF]
        gsems = bufs_and_sems[2 * NBUF:3 * NBUF]
        ssems = bufs_and_sems[3 * NBUF:]

        wid = lax.axis_index("s") * NC + lax.axis_index("c")
        for a in range(sb):
            pltpu.sync_copy(tok_hbm.at[a, wid], idx_v.at[a])

        iota = lax.iota(jnp.int32, 16)
        tds = [(iota + c * 16) // 8 for c in range(D // L)]
        rds = [(iota + c * 16) % 8 for c in range(D // L)]

        def fire_gather(b, s):
            pltpu.async_copy(
                emb_hbm.at[idx_v.at[s // 8, s % 8]], a_bufs[b], gsems[b])

        def wait_gather(b, s):
            pltpu.make_async_copy(
                emb_hbm.at[idx_v.at[s // 8, s % 8]], a_bufs[b],
                gsems[b]).wait()

        def fire_store(b, s):
            pltpu.async_copy(
                b_bufs[b].at[:, :, pl.ds(0, 128)],
                out_hbm.at[s, :, wid], ssems[b])

        def wait_store(b, s):
            pltpu.make_async_copy(
                b_bufs[b].at[:, :, pl.ds(0, 128)],
                out_hbm.at[s, :, wid], ssems[b]).wait()

        def transpose_scale(b):
            src, dst = a_bufs[b], b_bufs[b]

            def rows(i, _):
                for rr in range(2):
                    row = i * 2 + rr
                    rb = lax.broadcast(row, (16,))
                    for c in range(D // L):
                        v = src[row, pl.ds(c * L, L)] * SCALE
                        plsc.store_scatter(dst, [tds[c], rds[c], rb], v)
                return 0

            lax.fori_loop(0, 64, rows, 0)

        for b in range(NBUF):
            fire_gather(b, b)

        for b in range(NBUF):
            wait_gather(b, b)
            transpose_scale(b)
            fire_gather(b, NBUF + b)
            fire_store(b, b)

        def step(t, _):
            for b in range(NBUF):
                s = t * NBUF + b
                wait_gather(b, s)
                wait_store(b, s - NBUF)
                transpose_scale(b)
                fire_gather(b, s + NBUF)
                fire_store(b, s)
            return 0

        lax.fori_loop(1, nt - 1, step, 0)

        for b in range(NBUF):
            s = (nt - 1) * NBUF + b
            wait_gather(b, s)
            wait_store(b, s - NBUF)
            transpose_scale(b)
            fire_store(b, s)

        for b in range(NBUF):
            wait_store(b, (nt - 1) * NBUF + b)

    return k(emb, tok4)


def kernel(tokens, embedding):
    bsz, seq = tokens.shape
    sb, tbn = seq // 8, bsz // 128
    tok4 = (tokens.astype(jnp.int32).T
            .reshape(sb, 8, tbn, 128).transpose(0, 2, 1, 3))
    y5 = _lookup(embedding, tok4)
    return jnp.transpose(y5, (2, 4, 0, 1, 3)).reshape(bsz, seq, D)
